# Initial kernel scaffold; baseline (speedup 1.0000x reference)
import jax
import jax.numpy as jnp
from jax.experimental import pallas as pl


def kernel(inputs, table):
    raise NotImplementedError("write your pallas kernel here")


# speedup vs baseline: 4.1609x; 4.1609x over previous
import functools

import jax
import jax.numpy as jnp
from jax import lax
from jax.experimental import pallas as pl
from jax.experimental.pallas import tpu as pltpu
from jax.experimental.pallas import tpu_sc as plsc

VOCAB = 128
EMBED = 32
NC = 2
NS = 16
NW = NC * NS


@functools.cache
def _build(B):
    b_per_w = B // NW
    chunk = 2000
    nchunk = b_per_w // chunk

    mesh = plsc.VectorSubcoreMesh(core_axis_name="c", subcore_axis_name="s")

    @functools.partial(
        pl.kernel,
        mesh=mesh,
        out_type=jax.ShapeDtypeStruct((B, EMBED), jnp.float32),
        scratch_types=[
            pltpu.VMEM((chunk,), jnp.int32),
            pltpu.VMEM((chunk, EMBED), jnp.float32),
            pltpu.SemaphoreType.DMA,
        ],
        compiler_params=pltpu.CompilerParams(use_tc_tiling_on_sc=False),
    )
    def emb(idx_hbm, table_hbm, out_hbm, idx_v, rows_v, sem):
        wid = lax.axis_index("s") * NC + lax.axis_index("c")
        base = wid * b_per_w
        for g in range(nchunk):
            off = base + g * chunk
            pltpu.sync_copy(idx_hbm.at[pl.ds(off, chunk)], idx_v)
            pltpu.async_copy(table_hbm.at[idx_v], rows_v, sem).wait()
            pltpu.sync_copy(rows_v, out_hbm.at[pl.ds(off, chunk)])

    return emb


def kernel(inputs, table):
    shape = inputs.shape
    idx = inputs.reshape(-1).astype(jnp.int32)
    out = _build(idx.shape[0])(idx, table)
    return out.reshape(*shape, EMBED)


# speedup vs baseline: 4.1673x; 1.0015x over previous
import functools

import jax
import jax.numpy as jnp
from jax import lax
from jax.experimental import pallas as pl
from jax.experimental.pallas import tpu as pltpu
from jax.experimental.pallas import tpu_sc as plsc

VOCAB = 128
EMBED = 32
NC = 2
NS = 16
NW = NC * NS
NBUF = 3


@functools.cache
def _build(B):
    b_per_w = B // NW
    chunk = 1000
    nchunk = b_per_w // chunk

    mesh = plsc.VectorSubcoreMesh(core_axis_name="c", subcore_axis_name="s")

    @functools.partial(
        pl.kernel,
        mesh=mesh,
        out_type=jax.ShapeDtypeStruct((B, EMBED), jnp.float32),
        scratch_types=[
            pltpu.VMEM((NBUF, chunk), jnp.int32),
            pltpu.VMEM((NBUF, chunk, EMBED), jnp.float32),
            pltpu.SemaphoreType.DMA((NBUF,)),
            pltpu.SemaphoreType.DMA((NBUF,)),
            pltpu.SemaphoreType.DMA((NBUF,)),
        ],
        compiler_params=pltpu.CompilerParams(use_tc_tiling_on_sc=False),
    )
    def emb(idx_hbm, table_hbm, out_hbm, idx_v, rows_v, idx_sems, gat_sems, out_sems):
        wid = lax.axis_index("s") * NC + lax.axis_index("c")
        base = wid * b_per_w

        def idx_copy(g, b):
            return pltpu.make_async_copy(
                idx_hbm.at[pl.ds(base + g * chunk, chunk)], idx_v.at[b],
                idx_sems.at[b])

        def gat_copy(b):
            return pltpu.make_async_copy(
                table_hbm.at[idx_v.at[b]], rows_v.at[b], gat_sems.at[b])

        def out_copy(g, b):
            return pltpu.make_async_copy(
                rows_v.at[b], out_hbm.at[pl.ds(base + g * chunk, chunk)],
                out_sems.at[b])

        for g in range(min(NBUF, nchunk)):
            idx_copy(g, g).start()
        for g in range(nchunk + 1):
            b = g % NBUF
            if g < nchunk:
                idx_copy(g, b).wait()
                if g >= NBUF:
                    out_copy(g - NBUF, b).wait()
                gat_copy(b).start()
            if g >= 1:
                pb = (g - 1) % NBUF
                gat_copy(pb).wait()
                out_copy(g - 1, pb).start()
                if g - 1 + NBUF < nchunk:
                    idx_copy(g - 1 + NBUF, pb).start()
        for g in range(max(nchunk - NBUF, 0), nchunk):
            out_copy(g, g % NBUF).wait()

    return emb


def kernel(inputs, table):
    shape = inputs.shape
    idx = inputs.reshape(-1).astype(jnp.int32)
    out = _build(idx.shape[0])(idx, table)
    return out.reshape(*shape, EMBED)


# speedup vs baseline: 7.8237x; 1.8774x over previous
import functools

import jax
import jax.numpy as jnp
from jax import lax
from jax.experimental import pallas as pl
from jax.experimental.pallas import tpu as pltpu
from jax.experimental.pallas import tpu_sc as plsc

VOCAB = 128
EMBED = 32
NC = 2
NS = 16
NW = NC * NS
NBUF = 3


@functools.cache
def _build(B):
    b_per_w = B // NW
    chunk = 1000
    nchunk = b_per_w // chunk

    mesh = plsc.VectorSubcoreMesh(core_axis_name="c", subcore_axis_name="s")

    @functools.partial(
        pl.kernel,
        mesh=mesh,
        out_type=jax.ShapeDtypeStruct((B, EMBED), jnp.float32),
        scratch_types=[
            pltpu.VMEM_SHARED((VOCAB, EMBED), jnp.float32),
            pltpu.VMEM((NBUF, chunk), jnp.int32),
            pltpu.VMEM((NBUF, chunk, EMBED), jnp.float32),
            pltpu.SemaphoreType.DMA((NBUF,)),
            pltpu.SemaphoreType.DMA((NBUF,)),
            pltpu.SemaphoreType.DMA((NBUF,)),
        ],
        compiler_params=pltpu.CompilerParams(use_tc_tiling_on_sc=False),
    )
    def emb(idx_hbm, table_hbm, out_hbm, table_sh, idx_v, rows_v, idx_sems,
            gat_sems, out_sems):
        wid = lax.axis_index("s") * NC + lax.axis_index("c")
        base = wid * b_per_w

        @pl.when(lax.axis_index("s") == 0)
        def _():
            pltpu.sync_copy(table_hbm, table_sh)

        plsc.subcore_barrier()

        def idx_copy(g, b):
            return pltpu.make_async_copy(
                idx_hbm.at[pl.ds(base + g * chunk, chunk)], idx_v.at[b],
                idx_sems.at[b])

        def gat_copy(b):
            return pltpu.make_async_copy(
                table_sh.at[idx_v.at[b]], rows_v.at[b], gat_sems.at[b])

        def out_copy(g, b):
            return pltpu.make_async_copy(
                rows_v.at[b], out_hbm.at[pl.ds(base + g * chunk, chunk)],
                out_sems.at[b])

        for g in range(min(NBUF, nchunk)):
            idx_copy(g, g).start()
        for g in range(nchunk + 1):
            b = g % NBUF
            if g < nchunk:
                idx_copy(g, b).wait()
                if g >= NBUF:
                    out_copy(g - NBUF, b).wait()
                gat_copy(b).start()
            if g >= 1:
                pb = (g - 1) % NBUF
                gat_copy(pb).wait()
                out_copy(g - 1, pb).start()
                if g - 1 + NBUF < nchunk:
                    idx_copy(g - 1 + NBUF, pb).start()
        for g in range(max(nchunk - NBUF, 0), nchunk):
            out_copy(g, g % NBUF).wait()

    return emb


def kernel(inputs, table):
    shape = inputs.shape
    idx = inputs.reshape(-1).astype(jnp.int32)
    out = _build(idx.shape[0])(idx, table)
    return out.reshape(*shape, EMBED)
